# baseline (speedup 1.0000x reference)
import functools

import jax
import jax.numpy as jnp
from jax import lax
from jax.experimental import pallas as pl
from jax.experimental.pallas import tpu as pltpu
from jax.experimental.pallas import tpu_sc as plsc

D = 64
SCALE = float(D) ** 0.5

NC = 2
NS = 16
NW = NC * NS
L = 16


def _body(chunk, n_chunks, table_hbm, idx_hbm, out_hbm,
          idx_v, rows_v, sem_rows):
    wid = lax.axis_index("s") * NC + lax.axis_index("c")
    b_per_w = chunk * n_chunks
    base = wid * b_per_w

    def step(g, _):
        off = base + g * chunk
        pltpu.sync_copy(idx_hbm.at[pl.ds(off, chunk)], idx_v)
        pltpu.async_copy(table_hbm.at[idx_v], rows_v, sem_rows).wait()

        def row_fn(r, _):
            for j in range(D // L):
                sl = (r, pl.ds(j * L, L))
                rows_v[sl] = rows_v[sl] * SCALE
            return 0
        lax.fori_loop(0, chunk, row_fn, 0, unroll=4)

        pltpu.sync_copy(rows_v, out_hbm.at[pl.ds(off, chunk)])
        return 0

    lax.fori_loop(0, n_chunks, step, 0)


@functools.partial(jax.jit, static_argnames=("chunk", "n_chunks"))
def _embed_sc(idx_flat, lookup_table, chunk, n_chunks):
    b_total = chunk * n_chunks * NW
    mesh = plsc.VectorSubcoreMesh(core_axis_name="c", subcore_axis_name="s")
    run = pl.kernel(
        functools.partial(_body, chunk, n_chunks),
        out_type=jax.ShapeDtypeStruct((b_total, D), jnp.float32),
        mesh=mesh,
        scratch_types=[
            pltpu.VMEM((chunk,), jnp.int32),
            pltpu.VMEM((chunk, D), jnp.float32),
            pltpu.SemaphoreType.DMA,
        ],
        compiler_params=pltpu.CompilerParams(use_tc_tiling_on_sc=False),
    )
    return run(lookup_table, idx_flat)


def kernel(inputs, lookup_table):
    B, T = inputs.shape
    idx_flat = inputs.reshape(B * T).astype(jnp.int32)
    n_total = B * T
    b_per_w = n_total // NW
    chunk = 512
    out = _embed_sc(idx_flat, lookup_table, chunk, b_per_w // chunk)
    return out.reshape(B, T, D)


# speedup vs baseline: 1.0866x; 1.0866x over previous
import functools

import jax
import jax.numpy as jnp
from jax import lax
from jax.experimental import pallas as pl
from jax.experimental.pallas import tpu as pltpu
from jax.experimental.pallas import tpu_sc as plsc

D = 64
SCALE = float(D) ** 0.5

NC = 2
NS = 16
NW = NC * NS
L = 16


def _body(chunk, n_chunks, table_hbm, idx_hbm, out_hbm,
          idx_v, rows_v, sem_idx, sem_rows, sem_out):
    wid = lax.axis_index("s") * NC + lax.axis_index("c")
    b_per_w = chunk * n_chunks
    base = wid * b_per_w

    def idx_copy(g, buf):
        return pltpu.make_async_copy(
            idx_hbm.at[pl.ds(base + g * chunk, chunk)], idx_v.at[buf],
            sem_idx)

    def gather_copy(buf):
        return pltpu.make_async_copy(table_hbm.at[idx_v.at[buf]],
                                     rows_v.at[buf], sem_rows)

    def out_copy(g, buf):
        return pltpu.make_async_copy(
            rows_v.at[buf], out_hbm.at[pl.ds(base + g * chunk, chunk)],
            sem_out)

    def scale_rows(buf):
        rb = rows_v.at[buf]

        def row_fn(r, _):
            for j in range(D // L):
                sl = (r, pl.ds(j * L, L))
                rb[sl] = rb[sl] * SCALE
            return 0
        lax.fori_loop(0, chunk, row_fn, 0, unroll=4)

    idx_copy(0, 0).start()
    idx_copy(1, 1).start()
    idx_copy(0, 0).wait()
    gather_copy(0).start()

    def step(g, _):
        buf = lax.rem(g, 2)
        nxt = 1 - buf

        @pl.when(g >= 1)
        def _():
            out_copy(g - 1, nxt).wait()

        gather_copy(buf).wait()

        @pl.when(g + 1 < n_chunks)
        def _():
            idx_copy(g + 1, nxt).wait()
            gather_copy(nxt).start()

        @pl.when(g + 2 < n_chunks)
        def _():
            idx_copy(g + 2, buf).start()

        scale_rows(buf)
        out_copy(g, buf).start()
        return 0

    lax.fori_loop(0, n_chunks, step, 0)
    out_copy(n_chunks - 1, lax.rem(n_chunks - 1, 2)).wait()


@functools.partial(jax.jit, static_argnames=("chunk", "n_chunks"))
def _embed_sc(idx_flat, lookup_table, chunk, n_chunks):
    b_total = chunk * n_chunks * NW
    mesh = plsc.VectorSubcoreMesh(core_axis_name="c", subcore_axis_name="s")
    run = pl.kernel(
        functools.partial(_body, chunk, n_chunks),
        out_type=jax.ShapeDtypeStruct((b_total, D), jnp.float32),
        mesh=mesh,
        scratch_types=[
            pltpu.VMEM((2, chunk), jnp.int32),
            pltpu.VMEM((2, chunk, D), jnp.float32),
            pltpu.SemaphoreType.DMA,
            pltpu.SemaphoreType.DMA,
            pltpu.SemaphoreType.DMA,
        ],
        compiler_params=pltpu.CompilerParams(use_tc_tiling_on_sc=False),
    )
    return run(lookup_table, idx_flat)


def kernel(inputs, lookup_table):
    B, T = inputs.shape
    idx_flat = inputs.reshape(B * T).astype(jnp.int32)
    n_total = B * T
    b_per_w = n_total // NW
    chunk = 512
    out = _embed_sc(idx_flat, lookup_table, chunk, b_per_w // chunk)
    return out.reshape(B, T, D)
